# baseline (speedup 1.0000x reference)
import jax
import jax.numpy as jnp
from jax import lax
from jax.experimental import pallas as pl
from jax.experimental.pallas import tpu as pltpu
from jax.experimental.pallas import tpu_sc as plsc

N = 10000
D_IN = 256
D_H = 512

NC = 2
NS = 16

NP = 10112
TPW = NP // NS
DUMMY = N

E = 160000
E2 = 163840
BE = 128
NBT = E2 // BE
NB_T = NBT // NS
NB_H = NB_T // 2


def _sc_segment_sum(C, W, with_counts):
  mesh = plsc.VectorSubcoreMesh(core_axis_name="c", subcore_axis_name="s")
  out_type = [jax.ShapeDtypeStruct((C, NP, W), jnp.float32)]
  if with_counts:
    assert W == 128
    out_type.append(jax.ShapeDtypeStruct((NP, 128), jnp.float32))

  scratch = [
      pltpu.VMEM((NB_H, BE), jnp.int32),
      pltpu.VMEM((NB_T, BE), jnp.int32),
      pltpu.VMEM((2, BE, W), jnp.float32),
      pltpu.VMEM_SHARED((NP, W), jnp.float32),
      pltpu.SemaphoreType.DMA,
      pltpu.SemaphoreType.DMA,
  ]

  def body(table, src_h, dst_h, zeros_w, *rest):
    if with_counts:
      out, cnt_out, src_b, dst_all, rows, acc, sem0, sem1 = rest
    else:
      out, src_b, dst_all, rows, acc, sem0, sem1 = rest
    cid = lax.axis_index("c")
    sid = lax.axis_index("s")
    bb = sid * NB_T
    row0 = sid * TPW

    def zero_acc():
      pltpu.sync_copy(zeros_w.at[pl.ds(row0, TPW)], acc.at[pl.ds(row0, TPW)])

    if with_counts:
      @pl.when(cid == 1)
      def _():
        pltpu.sync_copy(dst_h.at[pl.ds(bb, NB_T)], dst_all)

        def init_ones(r, carry):
          for kk in range(W // 16):
            rows[0, r, pl.ds(kk * 16, 16)] = jnp.ones((16,), jnp.float32)
          return carry
        lax.fori_loop(0, BE, init_ones, 0)
        zero_acc()
        plsc.subcore_barrier()

        def cnt_step(j, carry):
          pltpu.sync_copy(rows.at[0], acc.at[dst_all.at[j]], add=True)
          return carry
        lax.fori_loop(0, NB_T, cnt_step, 0)
        plsc.subcore_barrier()
        pltpu.sync_copy(acc.at[pl.ds(row0, TPW)],
                        cnt_out.at[pl.ds(row0, TPW)])

    @pl.when(cid == 0)
    def _():
      pltpu.sync_copy(dst_h.at[pl.ds(bb, NB_T)], dst_all)
      for c in range(C):
        with jax.named_scope("zero"):
          zero_acc()
          plsc.subcore_barrier()

        with jax.named_scope("edges"):
          for half in range(2):
            hb = half * NB_H
            pltpu.sync_copy(src_h.at[pl.ds(bb + hb, NB_H)], src_b)
            if c > 0:
              def shift_row(j, carry, c=c):
                for kk in range(BE // 16):
                  src_b[j, pl.ds(kk * 16, 16)] = (
                      src_b[j, pl.ds(kk * 16, 16)] + jnp.int32(c * N))
                return carry
              lax.fori_loop(0, NB_H, shift_row, 0)

            def pair_step(i, carry, hb=hb):
              j0 = 2 * i
              j1 = 2 * i + 1
              d0 = pltpu.async_copy(table.at[src_b.at[j0]],
                                    rows.at[0], sem0)
              d1 = pltpu.async_copy(table.at[src_b.at[j1]],
                                    rows.at[1], sem1)
              d0.wait()
              pltpu.sync_copy(rows.at[0], acc.at[dst_all.at[hb + j0]],
                              add=True)
              d1.wait()
              pltpu.sync_copy(rows.at[1], acc.at[dst_all.at[hb + j1]],
                              add=True)
              return carry
            lax.fori_loop(0, NB_H // 2, pair_step, 0)
          plsc.subcore_barrier()
        with jax.named_scope("writeout"):
          pltpu.sync_copy(acc.at[pl.ds(row0, TPW)],
                          out.at[c, pl.ds(row0, TPW)])
        if c + 1 < C:
          plsc.subcore_barrier()

  return pl.kernel(body, out_type=out_type, mesh=mesh, scratch_types=scratch)


BN = 400


def _tc_sage_layer(agg, cnt, x_chunks, wl_t, wr_t, b, c_in, relu):
  d_in = c_in * 128
  c_out = wl_t.shape[1] // 128

  def body(agg_ref, cnt_ref, x_ref, wl_ref, wr_ref, b_ref, o_ref):
    aggf = jnp.concatenate([agg_ref[i] for i in range(c_in)], axis=1)
    cntc = cnt_ref[:, 0:1]
    mean = aggf / jnp.maximum(cntc, 1.0)
    xf = jnp.concatenate([x_ref[i] for i in range(c_in)], axis=1)
    h = (jnp.dot(mean, wl_ref[...], preferred_element_type=jnp.float32)
         + jnp.dot(xf, wr_ref[...], preferred_element_type=jnp.float32)
         + b_ref[0, 0])
    if relu:
      h = jnp.maximum(h, 0.0)
    o_ref[0] = h

  return pl.pallas_call(
      body,
      grid=(c_out, N // BN),
      in_specs=[
          pl.BlockSpec((c_in, BN, 128), lambda c, i: (0, i, 0)),
          pl.BlockSpec((BN, 128), lambda c, i: (i, 0)),
          pl.BlockSpec((c_in, BN, 128), lambda c, i: (0, i, 0)),
          pl.BlockSpec((d_in, 128), lambda c, i: (0, c)),
          pl.BlockSpec((d_in, 128), lambda c, i: (0, c)),
          pl.BlockSpec((1, 1, 128), lambda c, i: (c, 0, 0)),
      ],
      out_specs=pl.BlockSpec((1, BN, 128), lambda c, i: (c, i, 0)),
      out_shape=jax.ShapeDtypeStruct((c_out, N, 128), jnp.float32),
  )(agg, cnt, x_chunks, wl_t, wr_t, b)


def _tc_project(h_chunks, wl2_t128, wr2_t16):
  def body(h_ref, wl_ref, wr_ref, zl_ref, zr_ref):
    hf = jnp.concatenate([h_ref[i] for i in range(4)], axis=1)
    zl_ref[...] = jnp.dot(hf, wl_ref[...], preferred_element_type=jnp.float32)
    zr_ref[...] = jnp.dot(hf, wr_ref[...], preferred_element_type=jnp.float32)

  return pl.pallas_call(
      body,
      grid=(N // BN,),
      in_specs=[
          pl.BlockSpec((4, BN, 128), lambda i: (0, i, 0)),
          pl.BlockSpec((D_H, 128), lambda i: (0, 0)),
          pl.BlockSpec((D_H, 16), lambda i: (0, 0)),
      ],
      out_specs=[
          pl.BlockSpec((BN, 128), lambda i: (i, 0)),
          pl.BlockSpec((BN, 16), lambda i: (i, 0)),
      ],
      out_shape=[
          jax.ShapeDtypeStruct((N, 128), jnp.float32),
          jax.ShapeDtypeStruct((N, 16), jnp.float32),
      ],
  )(h_chunks, wl2_t128, wr2_t16)


def _tc_final(z_agg, cnt, zr, b2_16):
  def body(zp_ref, cnt_ref, zr_ref, b_ref, o_ref):
    zagg = zp_ref[0][:, 0:16]
    cntc = cnt_ref[:, 0:16]
    mean = zagg / jnp.maximum(cntc, 1.0)
    o_ref[...] = jax.nn.sigmoid(mean + zr_ref[...] + b_ref[0])

  return pl.pallas_call(
      body,
      grid=(N // BN,),
      in_specs=[
          pl.BlockSpec((1, BN, 128), lambda i: (0, i, 0)),
          pl.BlockSpec((BN, 128), lambda i: (i, 0)),
          pl.BlockSpec((BN, 16), lambda i: (i, 0)),
          pl.BlockSpec((1, 16), lambda i: (0, 0)),
      ],
      out_specs=pl.BlockSpec((BN, 16), lambda i: (i, 0)),
      out_shape=jax.ShapeDtypeStruct((N, 16), jnp.float32),
  )(z_agg, cnt, zr, b2_16)


def kernel(x, edge_index, Wl0, Wr0, b0, Wl1, Wr1, b1, Wl2, Wr2, b2):
  pad = E2 - E
  src = jnp.concatenate(
      [edge_index[0], jnp.zeros((pad,), jnp.int32)]).reshape(NBT, BE)
  dst = jnp.concatenate(
      [edge_index[1], jnp.full((pad,), DUMMY, jnp.int32)]).reshape(NBT, BE)
  x_flat = x.reshape(N, 2, 128).transpose(1, 0, 2).reshape(2 * N, 128)
  x_chunks = x_flat.reshape(2, N, 128)
  zeros128 = jnp.zeros((NP, 128), jnp.float32)
  wl0_t = Wl0.T
  wr0_t = Wr0.T
  b0_r = b0.reshape(4, 1, 128)
  wl1_t = Wl1.T
  wr1_t = Wr1.T
  b1_r = b1.reshape(4, 1, 128)
  wl2_t128 = jnp.pad(Wl2.T, ((0, 0), (0, 127)))
  wr2_t16 = jnp.pad(Wr2.T, ((0, 0), (0, 15)))
  b2_16 = jnp.broadcast_to(b2.reshape(1, 1), (1, 16))

  agg0, cnt = _sc_segment_sum(2, 128, True)(x_flat, src, dst, zeros128)
  h0 = _tc_sage_layer(agg0, cnt, x_chunks, wl0_t, wr0_t, b0_r,
                      c_in=2, relu=True)

  (agg1,) = _sc_segment_sum(4, 128, False)(
      h0.reshape(4 * N, 128), src, dst, zeros128)
  h1 = _tc_sage_layer(agg1, cnt, h0, wl1_t, wr1_t, b1_r,
                      c_in=4, relu=True)

  zl, zr = _tc_project(h1, wl2_t128, wr2_t16)
  (z_agg,) = _sc_segment_sum(1, 128, False)(zl, src, dst, zeros128)
  out16 = _tc_final(z_agg, cnt, zr, b2_16)
  return out16[:, 0:1]


# speedup vs baseline: 1.2664x; 1.2664x over previous
import jax
import jax.numpy as jnp
from jax import lax
from jax.experimental import pallas as pl
from jax.experimental.pallas import tpu as pltpu
from jax.experimental.pallas import tpu_sc as plsc

N = 10000
D_IN = 256
D_H = 512

NC = 2
NS = 16

NP = 10112
TPW = NP // NS
DUMMY = N

E = 160000
E2 = 163840
BE = 128
NBT = E2 // BE
NB0 = 1024
NB1 = NBT - NB0
NB_T0 = NB0 // NS
NB_T1 = NB1 // NS


def _sc_segment_sum(C, W, with_counts):
  mesh = plsc.VectorSubcoreMesh(core_axis_name="c", subcore_axis_name="s")
  out_type = [jax.ShapeDtypeStruct((NC, C, NP, W), jnp.float32)]
  if with_counts:
    assert W == 128
    out_type.append(jax.ShapeDtypeStruct((NC, NP, 128), jnp.float32))

  scratch = [
      pltpu.VMEM((NB_T0, BE), jnp.int32),
      pltpu.VMEM((NB_T0, BE), jnp.int32),
      pltpu.VMEM((2, BE, W), jnp.float32),
      pltpu.VMEM_SHARED((NP, W), jnp.float32),
      pltpu.SemaphoreType.DMA,
      pltpu.SemaphoreType.DMA,
  ]

  def body(table, src_h, dst_h, zeros_w, *rest):
    if with_counts:
      out, cnt_out, src_all, dst_all, rows, acc, sem0, sem1 = rest
    else:
      out, src_all, dst_all, rows, acc, sem0, sem1 = rest
    cid = lax.axis_index("c")
    sid = lax.axis_index("s")
    b0 = jnp.where(cid == 0, sid * NB_T0, NB0 + sid * NB_T1)
    nb = jnp.where(cid == 0, NB_T0, NB_T1)
    np2 = jnp.where(cid == 0, NB_T0 // 2, NB_T1 // 2)
    row0 = sid * TPW

    @pl.when(cid == 0)
    def _():
      pltpu.sync_copy(src_h.at[pl.ds(b0, NB_T0)], src_all)
      pltpu.sync_copy(dst_h.at[pl.ds(b0, NB_T0)], dst_all)

    @pl.when(cid == 1)
    def _():
      pltpu.sync_copy(src_h.at[pl.ds(b0, NB_T1)],
                      src_all.at[pl.ds(0, NB_T1)])
      pltpu.sync_copy(dst_h.at[pl.ds(b0, NB_T1)],
                      dst_all.at[pl.ds(0, NB_T1)])

    def zero_acc():
      pltpu.sync_copy(zeros_w.at[pl.ds(row0, TPW)], acc.at[pl.ds(row0, TPW)])

    if with_counts:
      def init_ones(r, carry):
        for kk in range(W // 16):
          rows[0, r, pl.ds(kk * 16, 16)] = jnp.ones((16,), jnp.float32)
        return carry
      lax.fori_loop(0, BE, init_ones, 0)
      zero_acc()
      plsc.subcore_barrier()

      def cnt_step(j, carry):
        pltpu.sync_copy(rows.at[0], acc.at[dst_all.at[j]], add=True)
        return carry
      lax.fori_loop(0, nb, cnt_step, 0)
      plsc.subcore_barrier()
      pltpu.sync_copy(acc.at[pl.ds(row0, TPW)],
                      cnt_out.at[cid, pl.ds(row0, TPW)])
      plsc.subcore_barrier()

    for c in range(C):
      if c > 0:
        def shift_row(j, carry):
          for kk in range(BE // 16):
            src_all[j, pl.ds(kk * 16, 16)] = (
                src_all[j, pl.ds(kk * 16, 16)] + jnp.int32(N))
          return carry
        lax.fori_loop(0, nb, shift_row, 0)

      with jax.named_scope("zero"):
        zero_acc()
        plsc.subcore_barrier()

      def pair_step(i, carry):
        j0 = 2 * i
        j1 = 2 * i + 1
        d0 = pltpu.async_copy(table.at[src_all.at[j0]], rows.at[0], sem0)
        d1 = pltpu.async_copy(table.at[src_all.at[j1]], rows.at[1], sem1)
        d0.wait()
        pltpu.sync_copy(rows.at[0], acc.at[dst_all.at[j0]], add=True)
        d1.wait()
        pltpu.sync_copy(rows.at[1], acc.at[dst_all.at[j1]], add=True)
        return carry

      with jax.named_scope("edges"):
        lax.fori_loop(0, np2, pair_step, 0)
        plsc.subcore_barrier()
      with jax.named_scope("writeout"):
        pltpu.sync_copy(acc.at[pl.ds(row0, TPW)],
                        out.at[cid, c, pl.ds(row0, TPW)])
      if c + 1 < C:
        plsc.subcore_barrier()

  return pl.kernel(body, out_type=out_type, mesh=mesh, scratch_types=scratch)


BN = 400


def _tc_sage_layer(parts, cnt_parts, x_chunks, wl_t, wr_t, b, c_in, relu):
  d_in = c_in * 128
  c_out = wl_t.shape[1] // 128

  def body(parts_ref, cnt_ref, x_ref, wl_ref, wr_ref, b_ref, o_ref):
    agg = parts_ref[0] + parts_ref[1]
    aggf = jnp.concatenate([agg[i] for i in range(c_in)], axis=1)
    cnt = cnt_ref[0, :, 0:1] + cnt_ref[1, :, 0:1]
    mean = aggf / jnp.maximum(cnt, 1.0)
    xf = jnp.concatenate([x_ref[i] for i in range(c_in)], axis=1)
    h = (jnp.dot(mean, wl_ref[...], preferred_element_type=jnp.float32)
         + jnp.dot(xf, wr_ref[...], preferred_element_type=jnp.float32)
         + b_ref[0, 0])
    if relu:
      h = jnp.maximum(h, 0.0)
    o_ref[0] = h

  return pl.pallas_call(
      body,
      grid=(c_out, N // BN),
      in_specs=[
          pl.BlockSpec((NC, c_in, BN, 128), lambda c, i: (0, 0, i, 0)),
          pl.BlockSpec((NC, BN, 128), lambda c, i: (0, i, 0)),
          pl.BlockSpec((c_in, BN, 128), lambda c, i: (0, i, 0)),
          pl.BlockSpec((d_in, 128), lambda c, i: (0, c)),
          pl.BlockSpec((d_in, 128), lambda c, i: (0, c)),
          pl.BlockSpec((1, 1, 128), lambda c, i: (c, 0, 0)),
      ],
      out_specs=pl.BlockSpec((1, BN, 128), lambda c, i: (c, i, 0)),
      out_shape=jax.ShapeDtypeStruct((c_out, N, 128), jnp.float32),
  )(parts, cnt_parts, x_chunks, wl_t, wr_t, b)


def _tc_project(h_chunks, wl2_t128, wr2_t16):
  def body(h_ref, wl_ref, wr_ref, zl_ref, zr_ref):
    hf = jnp.concatenate([h_ref[i] for i in range(4)], axis=1)
    zl_ref[...] = jnp.dot(hf, wl_ref[...], preferred_element_type=jnp.float32)
    zr_ref[...] = jnp.dot(hf, wr_ref[...], preferred_element_type=jnp.float32)

  return pl.pallas_call(
      body,
      grid=(N // BN,),
      in_specs=[
          pl.BlockSpec((4, BN, 128), lambda i: (0, i, 0)),
          pl.BlockSpec((D_H, 128), lambda i: (0, 0)),
          pl.BlockSpec((D_H, 16), lambda i: (0, 0)),
      ],
      out_specs=[
          pl.BlockSpec((BN, 128), lambda i: (i, 0)),
          pl.BlockSpec((BN, 16), lambda i: (i, 0)),
      ],
      out_shape=[
          jax.ShapeDtypeStruct((N, 128), jnp.float32),
          jax.ShapeDtypeStruct((N, 16), jnp.float32),
      ],
  )(h_chunks, wl2_t128, wr2_t16)


def _tc_final(z_parts, cnt_parts, zr, b2_16):
  def body(zp_ref, cnt_ref, zr_ref, b_ref, o_ref):
    zagg = (zp_ref[0, 0] + zp_ref[1, 0])[:, 0:16]
    cnt = (cnt_ref[0] + cnt_ref[1])[:, 0:16]
    mean = zagg / jnp.maximum(cnt, 1.0)
    o_ref[...] = jax.nn.sigmoid(mean + zr_ref[...] + b_ref[0])

  return pl.pallas_call(
      body,
      grid=(N // BN,),
      in_specs=[
          pl.BlockSpec((NC, 1, BN, 128), lambda i: (0, 0, i, 0)),
          pl.BlockSpec((NC, BN, 128), lambda i: (0, i, 0)),
          pl.BlockSpec((BN, 16), lambda i: (i, 0)),
          pl.BlockSpec((1, 16), lambda i: (0, 0)),
      ],
      out_specs=pl.BlockSpec((BN, 16), lambda i: (i, 0)),
      out_shape=jax.ShapeDtypeStruct((N, 16), jnp.float32),
  )(z_parts, cnt_parts, zr, b2_16)


def kernel(x, edge_index, Wl0, Wr0, b0, Wl1, Wr1, b1, Wl2, Wr2, b2):
  pad = E2 - E
  src = jnp.concatenate(
      [edge_index[0], jnp.zeros((pad,), jnp.int32)]).reshape(NBT, BE)
  dst = jnp.concatenate(
      [edge_index[1], jnp.full((pad,), DUMMY, jnp.int32)]).reshape(NBT, BE)
  x_flat = x.reshape(N, 2, 128).transpose(1, 0, 2).reshape(2 * N, 128)
  x_chunks = x_flat.reshape(2, N, 128)
  zeros128 = jnp.zeros((NP, 128), jnp.float32)
  wl0_t = Wl0.T
  wr0_t = Wr0.T
  b0_r = b0.reshape(4, 1, 128)
  wl1_t = Wl1.T
  wr1_t = Wr1.T
  b1_r = b1.reshape(4, 1, 128)
  wl2_t128 = jnp.pad(Wl2.T, ((0, 0), (0, 127)))
  wr2_t16 = jnp.pad(Wr2.T, ((0, 0), (0, 15)))
  b2_16 = jnp.broadcast_to(b2.reshape(1, 1), (1, 16))

  agg0_parts, cnt_parts = _sc_segment_sum(2, 128, True)(
      x_flat, src, dst, zeros128)
  h0 = _tc_sage_layer(agg0_parts, cnt_parts, x_chunks, wl0_t, wr0_t, b0_r,
                      c_in=2, relu=True)

  (agg1_parts,) = _sc_segment_sum(4, 128, False)(
      h0.reshape(4 * N, 128), src, dst, zeros128)
  h1 = _tc_sage_layer(agg1_parts, cnt_parts, h0, wl1_t, wr1_t, b1_r,
                      c_in=4, relu=True)

  zl, zr = _tc_project(h1, wl2_t128, wr2_t16)
  (z_parts,) = _sc_segment_sum(1, 128, False)(zl, src, dst, zeros128)
  out16 = _tc_final(z_parts, cnt_parts, zr, b2_16)
  return out16[:, 0:1]
